# baseline (speedup 1.0000x reference)
import functools

import jax
import jax.numpy as jnp
from jax import lax
from jax.experimental import pallas as pl
from jax.experimental.pallas import tpu as pltpu
from jax.experimental.pallas import tpu_sc as plsc

_D = 64
_CHUNK = 100
_NBUF = 16
_LANES = 16


def _sc_body(idx_hbm, table_hbm, out_hbm, idx_v, rows_v, out_v,
             *sems, nc, cpw, rpw):
    wid = lax.axis_index("s") * nc + lax.axis_index("c")

    pltpu.sync_copy(idx_hbm.at[pl.ds(wid * cpw, cpw)], idx_v)

    def fire(j, slot):
        pltpu.async_copy(table_hbm.at[idx_v.at[j]], rows_v.at[slot], sems[slot])

    for slot in range(_NBUF):
        fire(slot, slot)

    zero = jnp.zeros((_LANES,), jnp.float32)
    ngrp = _D // _LANES

    def reduce_chunk(slot, accs):
        def step(s, a):
            a0, a1, a2, a3 = a
            w0 = rows_v[slot, s, pl.ds(0, _LANES)]
            w1 = rows_v[slot, s, pl.ds(_LANES, _LANES)]
            u0 = lax.bitcast_convert_type(w0, jnp.uint32)
            u1 = lax.bitcast_convert_type(w1, jnp.uint32)
            msk = jnp.uint32(0xFFFF0000)
            a0 = a0 + lax.bitcast_convert_type(u0 << 16, jnp.float32)
            a2 = a2 + lax.bitcast_convert_type(u0 & msk, jnp.float32)
            a1 = a1 + lax.bitcast_convert_type(u1 << 16, jnp.float32)
            a3 = a3 + lax.bitcast_convert_type(u1 & msk, jnp.float32)
            return (a0, a1, a2, a3)
        return lax.fori_loop(0, _CHUNK, step, accs, unroll=10)

    rows_per_group = _NBUF // 2

    def group_body(g, carry):
        for r2 in range(rows_per_group):
            accs = (zero,) * ngrp
            for h in range(2):
                slot = 2 * r2 + h
                j = _NBUF * g + slot
                pltpu.make_async_copy(table_hbm.at[idx_v.at[j]],
                                      rows_v.at[slot], sems[slot]).wait()
                accs = reduce_chunk(slot, accs)
                nj = j + _NBUF

                @pl.when(nj < cpw)
                def _(nj=nj, slot=slot):
                    fire(nj, slot)

            row = rows_per_group * g + r2
            for c in range(ngrp):
                out_v[row, pl.ds(_LANES * c, _LANES)] = accs[c]
        return carry

    lax.fori_loop(0, rpw // rows_per_group, group_body, 0)
    pltpu.sync_copy(out_v, out_hbm.at[pl.ds(wid * rpw, rpw)])


def _sc_pool(idx2, table):
    info = plsc.get_sparse_core_info()
    nc, ns = info.num_cores, info.num_subcores
    nw = nc * ns
    nch = idx2.shape[0]
    bsz = nch * _CHUNK // 200
    cpw = nch // nw
    rpw = bsz // nw
    mesh = plsc.VectorSubcoreMesh(core_axis_name="c", subcore_axis_name="s")
    kfn = pl.kernel(
        functools.partial(_sc_body, nc=nc, cpw=cpw, rpw=rpw),
        mesh=mesh,
        out_type=jax.ShapeDtypeStruct((bsz, _D), jnp.float32),
        scratch_types=[
            pltpu.VMEM((cpw, _CHUNK), jnp.int32),
            pltpu.VMEM((_NBUF, _CHUNK, _D // 2), jnp.float32),
            pltpu.VMEM((rpw, _D), jnp.float32),
        ] + [pltpu.SemaphoreType.DMA] * _NBUF,
        compiler_params=pltpu.CompilerParams(use_tc_tiling_on_sc=False),
    )
    return kfn(idx2, table)


_TBK = 32768


def _tc_linearize(tt, enc_t):
    v = tt.shape[1]
    grid = (v + _TBK - 1) // _TBK
    q = _TBK // 4

    def body(in_ref, enc_ref, out_ref, idx_ref):
        @pl.when(pl.program_id(0) == 0)
        def _():
            idx_ref[...] = _remap_idx(enc_ref[...])

        x = in_ref[...]
        u = lax.bitcast_convert_type(x, jnp.uint32)
        b = (u + 0x7FFF + ((u >> 16) & 1)) >> 16
        parts = []
        for g in range(4):
            bg = b[:, g * q:(g + 1) * q]
            parts.append((bg[32:64, :] << 16) | bg[0:32, :])
        stack = jnp.concatenate(parts, axis=0)
        out_ref[...] = lax.bitcast_convert_type(stack, jnp.float32).T

    seq, bsz = enc_t.shape
    return pl.pallas_call(
        body,
        grid=(grid,),
        in_specs=[pl.BlockSpec((64, _TBK), lambda i: (0, i)),
                  pl.BlockSpec((seq, bsz), lambda i: (0, 0))],
        out_specs=[pl.BlockSpec((q, 128), lambda i: (i, 0)),
                   pl.BlockSpec((seq, bsz), lambda i: (0, 0))],
        out_shape=[jax.ShapeDtypeStruct((grid * q, 128), jnp.float32),
                   jax.ShapeDtypeStruct((seq, bsz), jnp.int32)],
    )(tt, enc_t)


def _remap_idx(enc):
    q = _TBK // 4
    i = enc // _TBK
    j = enc % _TBK
    return (i * q + j % q) * 4 + j // q


def _tc_head(pooled, mask_t, catv_t, c0, c1, c2, w, b):
    bsz = pooled.shape[0]
    ncls = w.shape[1]

    def body(pooled_ref, mask_ref, catv_ref, c0_ref, c1_ref, c2_ref,
             w_ref, b_ref, out_ref):
        denom = jnp.clip(jnp.sum(mask_ref[...], axis=0, keepdims=True),
                         1.0, None)
        xt = pooled_ref[...].T / denom
        cv = catv_ref[...]
        for col, cref in ((0, c0_ref), (1, c1_ref), (2, c2_ref)):
            n = cref.shape[0]
            oh = (lax.broadcasted_iota(jnp.int32, (n, bsz), 0)
                  == cv[col:col + 1, :]).astype(jnp.float32)
            xt = xt + jnp.dot(cref[...].T, oh,
                              preferred_element_type=jnp.float32)
        out_ref[...] = (jnp.dot(w_ref[...].T, xt,
                                preferred_element_type=jnp.float32)
                        + b_ref[...].T)

    return pl.pallas_call(
        body,
        out_shape=jax.ShapeDtypeStruct((ncls, bsz), jnp.float32),
    )(pooled, mask_t, catv_t, c0, c1, c2, w, b.reshape(1, ncls))


def kernel(encoded_text, attention_mask, categorical_vars, emb_table,
           cat_emb0, cat_emb1, cat_emb2, W, b):
    bsz, seq = encoded_text.shape
    tlin2, idxt = _tc_linearize(emb_table.T, encoded_text.astype(jnp.int32).T)
    tlin = tlin2.reshape(tlin2.shape[0] * 4, 32)
    idx2 = idxt.T.reshape(bsz * seq // _CHUNK, _CHUNK)
    pooled = _sc_pool(idx2, tlin)
    logits_t = _tc_head(pooled, attention_mask.T, categorical_vars.T,
                        cat_emb0, cat_emb1, cat_emb2, W, b)
    return logits_t.T


# speedup vs baseline: 1.0276x; 1.0276x over previous
import functools

import jax
import jax.numpy as jnp
from jax import lax
from jax.experimental import pallas as pl
from jax.experimental.pallas import tpu as pltpu
from jax.experimental.pallas import tpu_sc as plsc

_D = 64
_CHUNK = 100
_NBUF = 8
_LANES = 16


def _sc_body(idx_hbm, table_hbm, out_hbm, idx_v, rows_v, out_v,
             *sems, nc, cpw, rpw):
    wid = lax.axis_index("s") * nc + lax.axis_index("c")

    pltpu.sync_copy(idx_hbm.at[pl.ds(wid * cpw, cpw)], idx_v)

    def fire(j, slot):
        pltpu.async_copy(table_hbm.at[idx_v.at[j]], rows_v.at[slot], sems[slot])

    for slot in range(_NBUF):
        fire(slot, slot)

    zero = jnp.zeros((_LANES,), jnp.float32)
    ngrp = _D // _LANES

    def reduce_chunk(slot, accs):
        def step(s, a):
            a0, a1, a2, a3 = a
            w0 = rows_v[slot, s, pl.ds(0, _LANES)]
            w1 = rows_v[slot, s, pl.ds(_LANES, _LANES)]
            u0 = lax.bitcast_convert_type(w0, jnp.uint32)
            u1 = lax.bitcast_convert_type(w1, jnp.uint32)
            a0 = a0 + lax.bitcast_convert_type(u0 << 16, jnp.float32)
            a2 = a2 + w0
            a1 = a1 + lax.bitcast_convert_type(u1 << 16, jnp.float32)
            a3 = a3 + w1
            return (a0, a1, a2, a3)
        return lax.fori_loop(0, _CHUNK, step, accs, unroll=10)

    rows_per_group = _NBUF // 2

    def group_body(g, carry):
        for r2 in range(rows_per_group):
            accs = (zero,) * ngrp
            for h in range(2):
                slot = 2 * r2 + h
                j = _NBUF * g + slot
                pltpu.make_async_copy(table_hbm.at[idx_v.at[j]],
                                      rows_v.at[slot], sems[slot]).wait()
                accs = reduce_chunk(slot, accs)
                nj = j + _NBUF

                @pl.when(nj < cpw)
                def _(nj=nj, slot=slot):
                    fire(nj, slot)

            row = rows_per_group * g + r2
            for c in range(ngrp):
                out_v[row, pl.ds(_LANES * c, _LANES)] = accs[c]
        return carry

    lax.fori_loop(0, rpw // rows_per_group, group_body, 0)
    pltpu.sync_copy(out_v, out_hbm.at[pl.ds(wid * rpw, rpw)])


def _sc_pool(idx2, table):
    info = plsc.get_sparse_core_info()
    nc, ns = info.num_cores, info.num_subcores
    nw = nc * ns
    nch = idx2.shape[0]
    bsz = nch * _CHUNK // 200
    cpw = nch // nw
    rpw = bsz // nw
    mesh = plsc.VectorSubcoreMesh(core_axis_name="c", subcore_axis_name="s")
    kfn = pl.kernel(
        functools.partial(_sc_body, nc=nc, cpw=cpw, rpw=rpw),
        mesh=mesh,
        out_type=jax.ShapeDtypeStruct((bsz, _D), jnp.float32),
        scratch_types=[
            pltpu.VMEM((cpw, _CHUNK), jnp.int32),
            pltpu.VMEM((_NBUF, _CHUNK, _D // 2), jnp.float32),
            pltpu.VMEM((rpw, _D), jnp.float32),
        ] + [pltpu.SemaphoreType.DMA] * _NBUF,
        compiler_params=pltpu.CompilerParams(use_tc_tiling_on_sc=False),
    )
    return kfn(idx2, table)


_TBK = 32768


def _tc_linearize(tt, enc_t):
    v = tt.shape[1]
    grid = (v + _TBK - 1) // _TBK
    q = _TBK // 4

    def body(in_ref, enc_ref, out_ref, idx_ref):
        @pl.when(pl.program_id(0) == 0)
        def _():
            idx_ref[...] = _remap_idx(enc_ref[...])

        x = in_ref[...]
        u = lax.bitcast_convert_type(x, jnp.uint32)
        b = (u + 0x7FFF + ((u >> 16) & 1)) >> 16
        parts = []
        for g in range(4):
            bg = b[:, g * q:(g + 1) * q]
            parts.append((bg[32:64, :] << 16) | bg[0:32, :])
        stack = jnp.concatenate(parts, axis=0)
        out_ref[...] = lax.bitcast_convert_type(stack, jnp.float32).T

    seq, bsz = enc_t.shape
    return pl.pallas_call(
        body,
        grid=(grid,),
        in_specs=[pl.BlockSpec((64, _TBK), lambda i: (0, i)),
                  pl.BlockSpec((seq, bsz), lambda i: (0, 0))],
        out_specs=[pl.BlockSpec((q, 128), lambda i: (i, 0)),
                   pl.BlockSpec((seq, bsz), lambda i: (0, 0))],
        out_shape=[jax.ShapeDtypeStruct((grid * q, 128), jnp.float32),
                   jax.ShapeDtypeStruct((seq, bsz), jnp.int32)],
    )(tt, enc_t)


def _remap_idx(enc):
    q = _TBK // 4
    i = enc // _TBK
    j = enc % _TBK
    return (i * q + j % q) * 4 + j // q


def _tc_head(pooled, mask_t, catv_t, c0, c1, c2, w, b):
    bsz = pooled.shape[0]
    ncls = w.shape[1]

    def body(pooled_ref, mask_ref, catv_ref, c0_ref, c1_ref, c2_ref,
             w_ref, b_ref, out_ref):
        denom = jnp.clip(jnp.sum(mask_ref[...], axis=0, keepdims=True),
                         1.0, None)
        xt = pooled_ref[...].T / denom
        cv = catv_ref[...]
        for col, cref in ((0, c0_ref), (1, c1_ref), (2, c2_ref)):
            n = cref.shape[0]
            oh = (lax.broadcasted_iota(jnp.int32, (n, bsz), 0)
                  == cv[col:col + 1, :]).astype(jnp.float32)
            xt = xt + jnp.dot(cref[...].T, oh,
                              preferred_element_type=jnp.float32)
        out_ref[...] = (jnp.dot(w_ref[...].T, xt,
                                preferred_element_type=jnp.float32)
                        + b_ref[...].T)

    return pl.pallas_call(
        body,
        out_shape=jax.ShapeDtypeStruct((ncls, bsz), jnp.float32),
    )(pooled, mask_t, catv_t, c0, c1, c2, w, b.reshape(1, ncls))


def kernel(encoded_text, attention_mask, categorical_vars, emb_table,
           cat_emb0, cat_emb1, cat_emb2, W, b):
    bsz, seq = encoded_text.shape
    tlin2, idxt = _tc_linearize(emb_table.T, encoded_text.astype(jnp.int32).T)
    tlin = tlin2.reshape(tlin2.shape[0] * 4, 32)
    idx2 = idxt.T.reshape(bsz * seq // _CHUNK, _CHUNK)
    pooled = _sc_pool(idx2, tlin)
    logits_t = _tc_head(pooled, attention_mask.T, categorical_vars.T,
                        cat_emb0, cat_emb1, cat_emb2, W, b)
    return logits_t.T
